# baseline (speedup 1.0000x reference)
import functools
import math

import jax
import jax.numpy as jnp
from jax import lax
from jax.experimental import pallas as pl
from jax.experimental.pallas import tpu as pltpu
from jax.experimental.pallas import tpu_sc as plsc

S, D, H, E, DFF = 2048, 768, 12, 8, 1536
DK = D // H
BT = 128
NB = 23
G = 3072
SQ = 512
HPACK = 12
LNEPS = 1e-5


def _attn_body(x_ref, wq_ref, wk_ref, wv_ref, bq_ref, bk_ref, bv_ref,
               o_ref, k_buf, v_buf):
    i = pl.program_id(1)

    @pl.when(i == 0)
    def _():
        xa = x_ref[...]
        k_buf[...] = jnp.dot(xa, wk_ref[...],
                             preferred_element_type=jnp.float32) + bk_ref[...]
        v_buf[...] = jnp.dot(xa, wv_ref[...],
                             preferred_element_type=jnp.float32) + bv_ref[...]

    xq = x_ref[pl.ds(i * SQ, SQ), :]
    qq = jnp.dot(xq, wq_ref[...],
                 preferred_element_type=jnp.float32) + bq_ref[...]
    for half in range(HPACK):
        sl = slice(half * DK, (half + 1) * DK)
        q = qq[:, sl]
        k = k_buf[:, sl]
        s = lax.dot_general(q, k, (((1,), (1,)), ((), ())),
                            preferred_element_type=jnp.float32)
        p = jnp.exp(s * (1.0 / math.sqrt(DK)))
        l = jnp.sum(p, axis=-1, keepdims=True)
        o = jnp.dot(p, v_buf[:, sl], preferred_element_type=jnp.float32)
        o_ref[:, sl] = o * (1.0 / l)


def _mid_body(x_ref, a_ref, wo_ref, bo_ref, g_ref, b_ref, gw_ref, gb_ref,
              x1_ref, dst_ref, bexp_ref, src_ref):
    o = jnp.dot(a_ref[...], wo_ref[...],
                preferred_element_type=jnp.float32) + bo_ref[...]
    t = x_ref[...] + o
    mean = jnp.mean(t, axis=-1, keepdims=True)
    var = jnp.mean((t - mean) ** 2, axis=-1, keepdims=True)
    x1 = (t - mean) * lax.rsqrt(var + LNEPS) * g_ref[...] + b_ref[...]
    x1_ref[...] = x1
    gs = jnp.dot(x1, gw_ref[...], preferred_element_type=jnp.float32) + gb_ref[...]
    mx = jnp.max(gs, axis=-1, keepdims=True)
    ii = lax.broadcasted_iota(jnp.int32, gs.shape, 1)
    eid = jnp.min(jnp.where(gs >= mx, ii, 128), axis=-1, keepdims=True)
    ie = lax.broadcasted_iota(jnp.int32, (S, E), 1)
    oh = (eid == ie).astype(jnp.float32)
    cs = oh
    sh = 1
    while sh < S:
        cs = cs + jnp.concatenate(
            [jnp.zeros((sh, E), jnp.float32), cs[:S - sh]], axis=0)
        sh *= 2
    rank = cs - oh
    myrank = jnp.sum(oh * rank, axis=1, keepdims=True)
    mypoff = jnp.zeros((S, 1), jnp.float32)
    bb = lax.broadcasted_iota(jnp.int32, (1, 128), 1).astype(jnp.float32) * BT
    bexp = jnp.zeros((1, 128), jnp.float32)
    poff = 0.0
    for ee in range(E):
        cnt = jnp.sum(oh[:, ee:ee + 1])
        padded = jnp.ceil(cnt / BT) * BT
        mypoff = mypoff + oh[:, ee:ee + 1] * poff
        end = poff + padded
        bexp = bexp + jnp.where(bb >= end, 1.0, 0.0)
        poff = end
    dst = (mypoff + myrank).astype(jnp.int32)
    dst_ref[...] = dst
    bexp_ref[...] = jnp.minimum(bexp, float(E - 1)).astype(jnp.int32)
    df = dst.astype(jnp.float32)
    ivec = lax.broadcasted_iota(jnp.int32, (S, 1), 0).astype(jnp.float32)
    for c in range(G // 512):
        gci = lax.broadcasted_iota(jnp.int32, (1, 512), 1) + c * 512
        ohg = jnp.where(df == gci.astype(jnp.float32), 1.0, 0.0)
        srcv = jnp.sum(ohg * ivec, axis=0, keepdims=True)
        anyv = jnp.sum(ohg, axis=0, keepdims=True)
        fall = jnp.bitwise_and(gci, S - 1)
        src_ref[:, c * 512:(c + 1) * 512] = jnp.where(
            anyv > 0.5, srcv.astype(jnp.int32), fall)


def _ffn_body(be_ref, xs_ref, w1_ref, b1_ref, w2_ref, b2_ref,
              g3_ref, b3_ref, ys_ref):
    e = be_ref[pl.program_id(0)]
    xb = xs_ref[...]
    h = jnp.dot(xb.astype(jnp.bfloat16), w1_ref[0].astype(jnp.bfloat16),
                preferred_element_type=jnp.float32) + b1_ref[0]
    gelu = 0.5 * h * (1.0 + lax.erf(h * (1.0 / math.sqrt(2.0))))
    silu = h * jax.nn.sigmoid(h)
    a = jnp.where(e % 2 == 0, gelu, silu)
    y = jnp.dot(a.astype(jnp.bfloat16), w2_ref[0].astype(jnp.bfloat16),
                preferred_element_type=jnp.float32) + b2_ref[0]
    t = xb + y
    mean = jnp.mean(t, axis=-1, keepdims=True)
    var = jnp.mean((t - mean) ** 2, axis=-1, keepdims=True)
    ys_ref[...] = (t - mean) * lax.rsqrt(var + LNEPS) * g3_ref[...] + b3_ref[...]


def _sc_gather(table, idx):
    n, d = table.shape
    b = idx.shape[0]
    info = plsc.get_sparse_core_info()
    nw = info.num_cores * info.num_subcores
    bpw = b // nw
    mesh = plsc.VectorSubcoreMesh(core_axis_name="c", subcore_axis_name="s")
    ch = 4
    cs = bpw // ch

    @functools.partial(
        pl.kernel, mesh=mesh,
        out_type=jax.ShapeDtypeStruct((b, d), jnp.float32),
        scratch_types=[
            pltpu.VMEM((ch, cs), jnp.int32),
            pltpu.VMEM((bpw, d), jnp.float32),
            pltpu.SemaphoreType.DMA((ch,)),
            pltpu.SemaphoreType.DMA((ch,)),
        ],
    )
    def k(table_hbm, idx_hbm, out_hbm, idx_v, rows_v, gsem, osem):
        wid = lax.axis_index("s") * info.num_cores + lax.axis_index("c")
        base = wid * bpw
        pltpu.sync_copy(idx_hbm.at[wid], idx_v)
        gets = [
            pltpu.async_copy(table_hbm.at[idx_v.at[c]],
                             rows_v.at[pl.ds(c * cs, cs)], gsem.at[c])
            for c in range(ch)
        ]
        puts = []
        for c in range(ch):
            gets[c].wait()
            puts.append(pltpu.async_copy(
                rows_v.at[pl.ds(c * cs, cs)],
                out_hbm.at[pl.ds(base + c * cs, cs)], osem.at[c]))
        for p in puts:
            p.wait()

    return k(table, idx.reshape(nw, ch, cs))


def kernel(x, Wq, bq, Wk, bk, Wv, bv, Wo, bo, ln1_g, ln1_b,
           gate_W, gate_b, eW1, eb1, eW2, eb2, ln3_g, ln3_b):
    f32 = jnp.float32
    x2 = x.reshape(S, D)
    hp = HPACK * DK

    attn = pl.pallas_call(
        _attn_body,
        grid=(H // HPACK, S // SQ),
        in_specs=[
            pl.BlockSpec((S, D), lambda p, i: (0, 0)),
            pl.BlockSpec((D, hp), lambda p, i: (0, p)),
            pl.BlockSpec((D, hp), lambda p, i: (0, p)),
            pl.BlockSpec((D, hp), lambda p, i: (0, p)),
            pl.BlockSpec((1, hp), lambda p, i: (0, p)),
            pl.BlockSpec((1, hp), lambda p, i: (0, p)),
            pl.BlockSpec((1, hp), lambda p, i: (0, p)),
        ],
        out_specs=pl.BlockSpec((SQ, hp), lambda p, i: (i, p)),
        out_shape=jax.ShapeDtypeStruct((S, D), f32),
        scratch_shapes=[
            pltpu.VMEM((S, hp), f32),
            pltpu.VMEM((S, hp), f32),
        ],
    )(x2, Wq, Wk, Wv, bq.reshape(1, D), bk.reshape(1, D), bv.reshape(1, D))

    gwp = jnp.zeros((D, 128), f32).at[:, :E].set(gate_W)
    gbp = jnp.full((1, 128), -1e30, f32).at[0, :E].set(gate_b)

    x1, dst, bexp, src = pl.pallas_call(
        _mid_body,
        out_shape=[
            jax.ShapeDtypeStruct((S, D), f32),
            jax.ShapeDtypeStruct((S, 1), jnp.int32),
            jax.ShapeDtypeStruct((1, 128), jnp.int32),
            jax.ShapeDtypeStruct((1, G), jnp.int32),
        ],
    )(x2, attn, Wo, bo.reshape(1, D),
      ln1_g.reshape(1, D), ln1_b.reshape(1, D), gwp, gbp)

    xs = _sc_gather(x1, src.reshape(G))

    ys = pl.pallas_call(
        _ffn_body,
        grid_spec=pltpu.PrefetchScalarGridSpec(
            num_scalar_prefetch=1,
            grid=(NB,),
            in_specs=[
                pl.BlockSpec((BT, D), lambda b, be: (b, 0)),
                pl.BlockSpec((1, D, DFF), lambda b, be: (be[b], 0, 0)),
                pl.BlockSpec((1, 1, DFF), lambda b, be: (be[b], 0, 0)),
                pl.BlockSpec((1, DFF, D), lambda b, be: (be[b], 0, 0)),
                pl.BlockSpec((1, 1, D), lambda b, be: (be[b], 0, 0)),
                pl.BlockSpec((1, D), lambda b, be: (0, 0)),
                pl.BlockSpec((1, D), lambda b, be: (0, 0)),
            ],
            out_specs=pl.BlockSpec((BT, D), lambda b, be: (b, 0)),
        ),
        out_shape=jax.ShapeDtypeStruct((NB * BT, D), f32),
    )(bexp.reshape(128)[:NB], xs, eW1, eb1.reshape(E, 1, DFF),
      eW2, eb2.reshape(E, 1, D),
      ln3_g.reshape(1, D), ln3_b.reshape(1, D))

    out = _sc_gather(ys, dst.reshape(S))
    return out.reshape(1, S, D)


# speedup vs baseline: 1.0154x; 1.0154x over previous
import functools
import math

import jax
import jax.numpy as jnp
from jax import lax
from jax.experimental import pallas as pl
from jax.experimental.pallas import tpu as pltpu
from jax.experimental.pallas import tpu_sc as plsc

S, D, H, E, DFF = 2048, 768, 12, 8, 1536
DK = D // H
BT = 128
NB = 23
G = 3072
SQ = 512
HPACK = 4
LNEPS = 1e-5


def _attn_body(x_ref, wq_ref, wk_ref, wv_ref, bq_ref, bk_ref, bv_ref,
               o_ref, k_buf, v_buf):
    i = pl.program_id(1)

    @pl.when(i == 0)
    def _():
        xa = x_ref[...]
        k_buf[...] = jnp.dot(xa, wk_ref[...],
                             preferred_element_type=jnp.float32) + bk_ref[...]
        v_buf[...] = jnp.dot(xa, wv_ref[...],
                             preferred_element_type=jnp.float32) + bv_ref[...]

    xq = x_ref[pl.ds(i * SQ, SQ), :]
    qq = jnp.dot(xq, wq_ref[...],
                 preferred_element_type=jnp.float32) + bq_ref[...]
    qq = qq * (1.0 / math.sqrt(DK))
    for half in range(HPACK):
        sl = slice(half * DK, (half + 1) * DK)
        q = qq[:, sl]
        k = k_buf[:, sl]
        s = lax.dot_general(q, k, (((1,), (1,)), ((), ())),
                            preferred_element_type=jnp.float32)
        p = jnp.exp(s)
        l = jnp.sum(p, axis=-1, keepdims=True)
        o = jnp.dot(p, v_buf[:, sl], preferred_element_type=jnp.float32)
        o_ref[:, sl] = o * (1.0 / l)


def _mid_body(x_ref, a_ref, wo_ref, bo_ref, g_ref, b_ref, gw_ref, gb_ref,
              x1_ref, dst_ref, bexp_ref, src_ref):
    o = jnp.dot(a_ref[...], wo_ref[...],
                preferred_element_type=jnp.float32) + bo_ref[...]
    t = x_ref[...] + o
    mean = jnp.mean(t, axis=-1, keepdims=True)
    var = jnp.mean((t - mean) ** 2, axis=-1, keepdims=True)
    x1 = (t - mean) * lax.rsqrt(var + LNEPS) * g_ref[...] + b_ref[...]
    x1_ref[...] = x1
    gs = jnp.dot(x1, gw_ref[...], preferred_element_type=jnp.float32) + gb_ref[...]
    mx = jnp.max(gs, axis=-1, keepdims=True)
    ii = lax.broadcasted_iota(jnp.int32, gs.shape, 1)
    eid = jnp.min(jnp.where(gs >= mx, ii, 128), axis=-1, keepdims=True)
    ie = lax.broadcasted_iota(jnp.int32, (S, E), 1)
    oh = (eid == ie).astype(jnp.float32)
    cs = oh
    sh = 1
    while sh < S:
        cs = cs + jnp.concatenate(
            [jnp.zeros((sh, E), jnp.float32), cs[:S - sh]], axis=0)
        sh *= 2
    rank = cs - oh
    myrank = jnp.sum(oh * rank, axis=1, keepdims=True)
    mypoff = jnp.zeros((S, 1), jnp.float32)
    bb = lax.broadcasted_iota(jnp.int32, (1, 128), 1).astype(jnp.float32) * BT
    bexp = jnp.zeros((1, 128), jnp.float32)
    poff = 0.0
    for ee in range(E):
        cnt = jnp.sum(oh[:, ee:ee + 1])
        padded = jnp.ceil(cnt / BT) * BT
        mypoff = mypoff + oh[:, ee:ee + 1] * poff
        end = poff + padded
        bexp = bexp + jnp.where(bb >= end, 1.0, 0.0)
        poff = end
    dst = (mypoff + myrank).astype(jnp.int32)
    dst_ref[...] = dst
    bexp_ref[...] = jnp.minimum(bexp, float(E - 1)).astype(jnp.int32)
    df = dst.astype(jnp.float32)
    ivec = lax.broadcasted_iota(jnp.int32, (S, 1), 0).astype(jnp.float32)
    for c in range(G // 512):
        gci = lax.broadcasted_iota(jnp.int32, (1, 512), 1) + c * 512
        ohg = jnp.where(df == gci.astype(jnp.float32), 1.0, 0.0)
        srcv = jnp.sum(ohg * ivec, axis=0, keepdims=True)
        anyv = jnp.sum(ohg, axis=0, keepdims=True)
        fall = jnp.bitwise_and(gci, S - 1)
        src_ref[:, c * 512:(c + 1) * 512] = jnp.where(
            anyv > 0.5, srcv.astype(jnp.int32), fall)


def _ffn_body(be_ref, xs_ref, w1_ref, b1_ref, w2_ref, b2_ref,
              g3_ref, b3_ref, ys_ref):
    e = be_ref[pl.program_id(0)]
    xb = xs_ref[...]
    h = jnp.dot(xb.astype(jnp.bfloat16), w1_ref[0].astype(jnp.bfloat16),
                preferred_element_type=jnp.float32) + b1_ref[0]
    gelu = 0.5 * h * (1.0 + lax.erf(h * (1.0 / math.sqrt(2.0))))
    silu = h * jax.nn.sigmoid(h)
    a = jnp.where(e % 2 == 0, gelu, silu)
    y = jnp.dot(a.astype(jnp.bfloat16), w2_ref[0].astype(jnp.bfloat16),
                preferred_element_type=jnp.float32) + b2_ref[0]
    t = xb + y
    mean = jnp.mean(t, axis=-1, keepdims=True)
    var = jnp.mean((t - mean) ** 2, axis=-1, keepdims=True)
    ys_ref[...] = (t - mean) * lax.rsqrt(var + LNEPS) * g3_ref[...] + b3_ref[...]


def _sc_gather(table, idx):
    n, d = table.shape
    b = idx.shape[0]
    info = plsc.get_sparse_core_info()
    nw = info.num_cores * info.num_subcores
    bpw = b // nw
    mesh = plsc.VectorSubcoreMesh(core_axis_name="c", subcore_axis_name="s")
    ch = 4
    cs = bpw // ch

    @functools.partial(
        pl.kernel, mesh=mesh,
        out_type=jax.ShapeDtypeStruct((b, d), jnp.float32),
        scratch_types=[
            pltpu.VMEM((ch, cs), jnp.int32),
            pltpu.VMEM((bpw, d), jnp.float32),
            pltpu.SemaphoreType.DMA((ch,)),
            pltpu.SemaphoreType.DMA((ch,)),
        ],
    )
    def k(table_hbm, idx_hbm, out_hbm, idx_v, rows_v, gsem, osem):
        wid = lax.axis_index("s") * info.num_cores + lax.axis_index("c")
        base = wid * bpw
        pltpu.sync_copy(idx_hbm.at[wid], idx_v)
        gets = [
            pltpu.async_copy(table_hbm.at[idx_v.at[c]],
                             rows_v.at[pl.ds(c * cs, cs)], gsem.at[c])
            for c in range(ch)
        ]
        puts = []
        for c in range(ch):
            gets[c].wait()
            puts.append(pltpu.async_copy(
                rows_v.at[pl.ds(c * cs, cs)],
                out_hbm.at[pl.ds(base + c * cs, cs)], osem.at[c]))
        for p in puts:
            p.wait()

    return k(table, idx.reshape(nw, ch, cs))


def kernel(x, Wq, bq, Wk, bk, Wv, bv, Wo, bo, ln1_g, ln1_b,
           gate_W, gate_b, eW1, eb1, eW2, eb2, ln3_g, ln3_b):
    f32 = jnp.float32
    x2 = x.reshape(S, D)
    hp = HPACK * DK

    attn = pl.pallas_call(
        _attn_body,
        grid=(H // HPACK, S // SQ),
        in_specs=[
            pl.BlockSpec((S, D), lambda p, i: (0, 0)),
            pl.BlockSpec((D, hp), lambda p, i: (0, p)),
            pl.BlockSpec((D, hp), lambda p, i: (0, p)),
            pl.BlockSpec((D, hp), lambda p, i: (0, p)),
            pl.BlockSpec((1, hp), lambda p, i: (0, p)),
            pl.BlockSpec((1, hp), lambda p, i: (0, p)),
            pl.BlockSpec((1, hp), lambda p, i: (0, p)),
        ],
        out_specs=pl.BlockSpec((SQ, hp), lambda p, i: (i, p)),
        out_shape=jax.ShapeDtypeStruct((S, D), f32),
        scratch_shapes=[
            pltpu.VMEM((S, hp), f32),
            pltpu.VMEM((S, hp), f32),
        ],
    )(x2, Wq, Wk, Wv, bq.reshape(1, D), bk.reshape(1, D), bv.reshape(1, D))

    gwp = jnp.zeros((D, 128), f32).at[:, :E].set(gate_W)
    gbp = jnp.full((1, 128), -1e30, f32).at[0, :E].set(gate_b)

    x1, dst, bexp, src = pl.pallas_call(
        _mid_body,
        out_shape=[
            jax.ShapeDtypeStruct((S, D), f32),
            jax.ShapeDtypeStruct((S, 1), jnp.int32),
            jax.ShapeDtypeStruct((1, 128), jnp.int32),
            jax.ShapeDtypeStruct((1, G), jnp.int32),
        ],
    )(x2, attn, Wo, bo.reshape(1, D),
      ln1_g.reshape(1, D), ln1_b.reshape(1, D), gwp, gbp)

    xs = _sc_gather(x1, src.reshape(G))

    ys = pl.pallas_call(
        _ffn_body,
        grid_spec=pltpu.PrefetchScalarGridSpec(
            num_scalar_prefetch=1,
            grid=(NB,),
            in_specs=[
                pl.BlockSpec((BT, D), lambda b, be: (b, 0)),
                pl.BlockSpec((1, D, DFF), lambda b, be: (be[b], 0, 0)),
                pl.BlockSpec((1, 1, DFF), lambda b, be: (be[b], 0, 0)),
                pl.BlockSpec((1, DFF, D), lambda b, be: (be[b], 0, 0)),
                pl.BlockSpec((1, 1, D), lambda b, be: (be[b], 0, 0)),
                pl.BlockSpec((1, D), lambda b, be: (0, 0)),
                pl.BlockSpec((1, D), lambda b, be: (0, 0)),
            ],
            out_specs=pl.BlockSpec((BT, D), lambda b, be: (b, 0)),
        ),
        out_shape=jax.ShapeDtypeStruct((NB * BT, D), f32),
    )(bexp.reshape(128)[:NB], xs, eW1, eb1.reshape(E, 1, DFF),
      eW2, eb2.reshape(E, 1, D),
      ln3_g.reshape(1, D), ln3_b.reshape(1, D))

    out = _sc_gather(ys, dst.reshape(S))
    return out.reshape(1, S, D)


# speedup vs baseline: 1.0160x; 1.0006x over previous
import functools
import math

import jax
import jax.numpy as jnp
from jax import lax
from jax.experimental import pallas as pl
from jax.experimental.pallas import tpu as pltpu
from jax.experimental.pallas import tpu_sc as plsc

S, D, H, E, DFF = 2048, 768, 12, 8, 1536
DK = D // H
BT = 128
NB = 23
G = 3072
SQ = 512
HPACK = 4
LNEPS = 1e-5


def _attn_body(x_ref, wq_ref, wk_ref, wv_ref, bq_ref, bk_ref, bv_ref,
               o_ref, k_buf, v_buf):
    i = pl.program_id(1)

    @pl.when(i == 0)
    def _():
        xa = x_ref[...]
        k_buf[...] = jnp.dot(xa, wk_ref[...], precision=lax.Precision.DEFAULT,
                             preferred_element_type=jnp.float32) + bk_ref[...]
        v_buf[...] = jnp.dot(xa, wv_ref[...], precision=lax.Precision.DEFAULT,
                             preferred_element_type=jnp.float32) + bv_ref[...]

    xq = x_ref[pl.ds(i * SQ, SQ), :]
    qq = jnp.dot(xq, wq_ref[...], precision=lax.Precision.DEFAULT,
                 preferred_element_type=jnp.float32) + bq_ref[...]
    qq = qq * (1.0 / math.sqrt(DK))
    for half in range(HPACK):
        sl = slice(half * DK, (half + 1) * DK)
        q = qq[:, sl]
        k = k_buf[:, sl]
        s = lax.dot_general(q, k, (((1,), (1,)), ((), ())),
                            precision=lax.Precision.DEFAULT,
                            preferred_element_type=jnp.float32)
        p = jnp.exp(s)
        l = jnp.sum(p, axis=-1, keepdims=True)
        o = jnp.dot(p, v_buf[:, sl], precision=lax.Precision.DEFAULT,
                    preferred_element_type=jnp.float32)
        o_ref[:, sl] = o * (1.0 / l)


def _mid_body(x_ref, a_ref, wo_ref, bo_ref, g_ref, b_ref, gw_ref, gb_ref,
              x1_ref, dst_ref, bexp_ref, src_ref):
    o = jnp.dot(a_ref[...], wo_ref[...], precision=lax.Precision.DEFAULT,
                preferred_element_type=jnp.float32) + bo_ref[...]
    t = x_ref[...] + o
    mean = jnp.mean(t, axis=-1, keepdims=True)
    var = jnp.mean((t - mean) ** 2, axis=-1, keepdims=True)
    x1 = (t - mean) * lax.rsqrt(var + LNEPS) * g_ref[...] + b_ref[...]
    x1_ref[...] = x1
    gs = jnp.dot(x1, gw_ref[...], preferred_element_type=jnp.float32) + gb_ref[...]
    mx = jnp.max(gs, axis=-1, keepdims=True)
    ii = lax.broadcasted_iota(jnp.int32, gs.shape, 1)
    eid = jnp.min(jnp.where(gs >= mx, ii, 128), axis=-1, keepdims=True)
    ie = lax.broadcasted_iota(jnp.int32, (S, E), 1)
    oh = (eid == ie).astype(jnp.float32)
    cs = oh
    sh = 1
    while sh < S:
        cs = cs + jnp.concatenate(
            [jnp.zeros((sh, E), jnp.float32), cs[:S - sh]], axis=0)
        sh *= 2
    rank = cs - oh
    myrank = jnp.sum(oh * rank, axis=1, keepdims=True)
    mypoff = jnp.zeros((S, 1), jnp.float32)
    bb = lax.broadcasted_iota(jnp.int32, (1, 128), 1).astype(jnp.float32) * BT
    bexp = jnp.zeros((1, 128), jnp.float32)
    poff = 0.0
    for ee in range(E):
        cnt = jnp.sum(oh[:, ee:ee + 1])
        padded = jnp.ceil(cnt / BT) * BT
        mypoff = mypoff + oh[:, ee:ee + 1] * poff
        end = poff + padded
        bexp = bexp + jnp.where(bb >= end, 1.0, 0.0)
        poff = end
    dst = (mypoff + myrank).astype(jnp.int32)
    dst_ref[...] = dst
    bexp_ref[...] = jnp.minimum(bexp, float(E - 1)).astype(jnp.int32)
    df = dst.astype(jnp.float32)
    ivec = lax.broadcasted_iota(jnp.int32, (S, 1), 0).astype(jnp.float32)
    for c in range(G // 512):
        gci = lax.broadcasted_iota(jnp.int32, (1, 512), 1) + c * 512
        ohg = jnp.where(df == gci.astype(jnp.float32), 1.0, 0.0)
        srcv = jnp.sum(ohg * ivec, axis=0, keepdims=True)
        anyv = jnp.sum(ohg, axis=0, keepdims=True)
        fall = jnp.bitwise_and(gci, S - 1)
        src_ref[:, c * 512:(c + 1) * 512] = jnp.where(
            anyv > 0.5, srcv.astype(jnp.int32), fall)


def _ffn_body(be_ref, xs_ref, w1_ref, b1_ref, w2_ref, b2_ref,
              g3_ref, b3_ref, ys_ref):
    e = be_ref[pl.program_id(0)]
    xb = xs_ref[...]
    h = jnp.dot(xb.astype(jnp.bfloat16), w1_ref[0].astype(jnp.bfloat16),
                preferred_element_type=jnp.float32) + b1_ref[0]
    gelu = 0.5 * h * (1.0 + lax.erf(h * (1.0 / math.sqrt(2.0))))
    silu = h * jax.nn.sigmoid(h)
    a = jnp.where(e % 2 == 0, gelu, silu)
    y = jnp.dot(a.astype(jnp.bfloat16), w2_ref[0].astype(jnp.bfloat16),
                preferred_element_type=jnp.float32) + b2_ref[0]
    t = xb + y
    mean = jnp.mean(t, axis=-1, keepdims=True)
    var = jnp.mean((t - mean) ** 2, axis=-1, keepdims=True)
    ys_ref[...] = (t - mean) * lax.rsqrt(var + LNEPS) * g3_ref[...] + b3_ref[...]


def _sc_gather(table, idx):
    n, d = table.shape
    b = idx.shape[0]
    info = plsc.get_sparse_core_info()
    nw = info.num_cores * info.num_subcores
    bpw = b // nw
    mesh = plsc.VectorSubcoreMesh(core_axis_name="c", subcore_axis_name="s")
    ch = 4
    cs = bpw // ch

    @functools.partial(
        pl.kernel, mesh=mesh,
        out_type=jax.ShapeDtypeStruct((b, d), jnp.float32),
        scratch_types=[
            pltpu.VMEM((ch, cs), jnp.int32),
            pltpu.VMEM((bpw, d), jnp.float32),
            pltpu.SemaphoreType.DMA((ch,)),
            pltpu.SemaphoreType.DMA((ch,)),
        ],
    )
    def k(table_hbm, idx_hbm, out_hbm, idx_v, rows_v, gsem, osem):
        wid = lax.axis_index("s") * info.num_cores + lax.axis_index("c")
        base = wid * bpw
        pltpu.sync_copy(idx_hbm.at[wid], idx_v)
        gets = [
            pltpu.async_copy(table_hbm.at[idx_v.at[c]],
                             rows_v.at[pl.ds(c * cs, cs)], gsem.at[c])
            for c in range(ch)
        ]
        puts = []
        for c in range(ch):
            gets[c].wait()
            puts.append(pltpu.async_copy(
                rows_v.at[pl.ds(c * cs, cs)],
                out_hbm.at[pl.ds(base + c * cs, cs)], osem.at[c]))
        for p in puts:
            p.wait()

    return k(table, idx.reshape(nw, ch, cs))


def kernel(x, Wq, bq, Wk, bk, Wv, bv, Wo, bo, ln1_g, ln1_b,
           gate_W, gate_b, eW1, eb1, eW2, eb2, ln3_g, ln3_b):
    f32 = jnp.float32
    x2 = x.reshape(S, D)
    hp = HPACK * DK

    attn = pl.pallas_call(
        _attn_body,
        grid=(H // HPACK, S // SQ),
        in_specs=[
            pl.BlockSpec((S, D), lambda p, i: (0, 0)),
            pl.BlockSpec((D, hp), lambda p, i: (0, p)),
            pl.BlockSpec((D, hp), lambda p, i: (0, p)),
            pl.BlockSpec((D, hp), lambda p, i: (0, p)),
            pl.BlockSpec((1, hp), lambda p, i: (0, p)),
            pl.BlockSpec((1, hp), lambda p, i: (0, p)),
            pl.BlockSpec((1, hp), lambda p, i: (0, p)),
        ],
        out_specs=pl.BlockSpec((SQ, hp), lambda p, i: (i, p)),
        out_shape=jax.ShapeDtypeStruct((S, D), f32),
        scratch_shapes=[
            pltpu.VMEM((S, hp), f32),
            pltpu.VMEM((S, hp), f32),
        ],
    )(x2, Wq, Wk, Wv, bq.reshape(1, D), bk.reshape(1, D), bv.reshape(1, D))

    gwp = jnp.zeros((D, 128), f32).at[:, :E].set(gate_W)
    gbp = jnp.full((1, 128), -1e30, f32).at[0, :E].set(gate_b)

    x1, dst, bexp, src = pl.pallas_call(
        _mid_body,
        out_shape=[
            jax.ShapeDtypeStruct((S, D), f32),
            jax.ShapeDtypeStruct((S, 1), jnp.int32),
            jax.ShapeDtypeStruct((1, 128), jnp.int32),
            jax.ShapeDtypeStruct((1, G), jnp.int32),
        ],
    )(x2, attn, Wo, bo.reshape(1, D),
      ln1_g.reshape(1, D), ln1_b.reshape(1, D), gwp, gbp)

    xs = _sc_gather(x1, src.reshape(G))

    ys = pl.pallas_call(
        _ffn_body,
        grid_spec=pltpu.PrefetchScalarGridSpec(
            num_scalar_prefetch=1,
            grid=(NB,),
            in_specs=[
                pl.BlockSpec((BT, D), lambda b, be: (b, 0)),
                pl.BlockSpec((1, D, DFF), lambda b, be: (be[b], 0, 0)),
                pl.BlockSpec((1, 1, DFF), lambda b, be: (be[b], 0, 0)),
                pl.BlockSpec((1, DFF, D), lambda b, be: (be[b], 0, 0)),
                pl.BlockSpec((1, 1, D), lambda b, be: (be[b], 0, 0)),
                pl.BlockSpec((1, D), lambda b, be: (0, 0)),
                pl.BlockSpec((1, D), lambda b, be: (0, 0)),
            ],
            out_specs=pl.BlockSpec((BT, D), lambda b, be: (b, 0)),
        ),
        out_shape=jax.ShapeDtypeStruct((NB * BT, D), f32),
    )(bexp.reshape(128)[:NB], xs, eW1, eb1.reshape(E, 1, DFF),
      eW2, eb2.reshape(E, 1, D),
      ln3_g.reshape(1, D), ln3_b.reshape(1, D))

    out = _sc_gather(ys, dst.reshape(S))
    return out.reshape(1, S, D)


# speedup vs baseline: 1.0505x; 1.0340x over previous
import functools
import math

import jax
import jax.numpy as jnp
from jax import lax
from jax.experimental import pallas as pl
from jax.experimental.pallas import tpu as pltpu
from jax.experimental.pallas import tpu_sc as plsc

S, D, H, E, DFF = 2048, 768, 12, 8, 1536
DK = D // H
BT = 128
NB = 23
G = 3072
SQ = 512
HPACK = 4
LNEPS = 1e-5


def _attn_body(x_ref, wq_ref, wk_ref, wv_ref, bq_ref, bk_ref, bv_ref,
               wo_ref, bo_ref, g_ref, b_ref, gw_ref, gb_ref,
               x1_ref, eid_ref, k_buf, v_buf):
    i = pl.program_id(0)

    @pl.when(i == 0)
    def _():
        xa = x_ref[...]
        k_buf[...] = jnp.dot(xa, wk_ref[...],
                             preferred_element_type=jnp.float32) + bk_ref[...]
        v_buf[...] = jnp.dot(xa, wv_ref[...],
                             preferred_element_type=jnp.float32) + bv_ref[...]

    xq = x_ref[pl.ds(i * SQ, SQ), :]
    qq = jnp.dot(xq, wq_ref[...],
                 preferred_element_type=jnp.float32) + bq_ref[...]
    qq = qq * (1.0 / math.sqrt(DK))
    outs = []
    for half in range(H):
        sl = slice(half * DK, (half + 1) * DK)
        q = qq[:, sl]
        k = k_buf[:, sl]
        s = lax.dot_general(q, k, (((1,), (1,)), ((), ())),
                            preferred_element_type=jnp.float32)
        p = jnp.exp(s)
        l = jnp.sum(p, axis=-1, keepdims=True)
        o = jnp.dot(p, v_buf[:, sl], preferred_element_type=jnp.float32)
        outs.append(o * (1.0 / l))
    attn = jnp.concatenate(outs, axis=1)
    o = jnp.dot(attn, wo_ref[...],
                preferred_element_type=jnp.float32) + bo_ref[...]
    t = xq + o
    mean = jnp.mean(t, axis=-1, keepdims=True)
    var = jnp.mean((t - mean) ** 2, axis=-1, keepdims=True)
    x1 = (t - mean) * lax.rsqrt(var + LNEPS) * g_ref[...] + b_ref[...]
    x1_ref[...] = x1
    gs = jnp.dot(x1, gw_ref[...], preferred_element_type=jnp.float32) + gb_ref[...]
    mx = jnp.max(gs, axis=-1, keepdims=True)
    ii = lax.broadcasted_iota(jnp.int32, gs.shape, 1)
    eid_ref[...] = jnp.min(jnp.where(gs >= mx, ii, 128), axis=-1, keepdims=True)


def _route_body(eid_ref, dst_ref, bexp_ref, src_ref):
    eid = eid_ref[...]
    ie = lax.broadcasted_iota(jnp.int32, (S, E), 1)
    oh = (eid == ie).astype(jnp.float32)
    cs = oh
    sh = 1
    while sh < S:
        cs = cs + jnp.concatenate(
            [jnp.zeros((sh, E), jnp.float32), cs[:S - sh]], axis=0)
        sh *= 2
    rank = cs - oh
    myrank = jnp.sum(oh * rank, axis=1, keepdims=True)
    mypoff = jnp.zeros((S, 1), jnp.float32)
    bb = lax.broadcasted_iota(jnp.int32, (1, 128), 1).astype(jnp.float32) * BT
    bexp = jnp.zeros((1, 128), jnp.float32)
    poff = 0.0
    for ee in range(E):
        cnt = jnp.sum(oh[:, ee:ee + 1])
        padded = jnp.ceil(cnt / BT) * BT
        mypoff = mypoff + oh[:, ee:ee + 1] * poff
        end = poff + padded
        bexp = bexp + jnp.where(bb >= end, 1.0, 0.0)
        poff = end
    dst = (mypoff + myrank).astype(jnp.int32)
    dst_ref[...] = dst
    bexp_ref[...] = jnp.minimum(bexp, float(E - 1)).astype(jnp.int32)
    df = dst.astype(jnp.float32)
    ivec = lax.broadcasted_iota(jnp.int32, (S, 1), 0).astype(jnp.float32)
    for c in range(G // 512):
        gci = lax.broadcasted_iota(jnp.int32, (1, 512), 1) + c * 512
        ohg = jnp.where(df == gci.astype(jnp.float32), 1.0, 0.0)
        srcv = jnp.sum(ohg * ivec, axis=0, keepdims=True)
        anyv = jnp.sum(ohg, axis=0, keepdims=True)
        fall = jnp.bitwise_and(gci, S - 1)
        src_ref[:, c * 512:(c + 1) * 512] = jnp.where(
            anyv > 0.5, srcv.astype(jnp.int32), fall)


def _ffn_body(be_ref, xs_ref, w1_ref, b1_ref, w2_ref, b2_ref,
              g3_ref, b3_ref, ys_ref):
    e = be_ref[pl.program_id(0)]
    xb = xs_ref[...]
    h = jnp.dot(xb.astype(jnp.bfloat16), w1_ref[0].astype(jnp.bfloat16),
                preferred_element_type=jnp.float32) + b1_ref[0]
    gelu = 0.5 * h * (1.0 + lax.erf(h * (1.0 / math.sqrt(2.0))))
    silu = h * jax.nn.sigmoid(h)
    a = jnp.where(e % 2 == 0, gelu, silu)
    y = jnp.dot(a.astype(jnp.bfloat16), w2_ref[0].astype(jnp.bfloat16),
                preferred_element_type=jnp.float32) + b2_ref[0]
    t = xb + y
    mean = jnp.mean(t, axis=-1, keepdims=True)
    var = jnp.mean((t - mean) ** 2, axis=-1, keepdims=True)
    ys_ref[...] = (t - mean) * lax.rsqrt(var + LNEPS) * g3_ref[...] + b3_ref[...]


def _sc_gather(table, idx):
    n, d = table.shape
    b = idx.shape[0]
    info = plsc.get_sparse_core_info()
    nw = info.num_cores * info.num_subcores
    bpw = b // nw
    mesh = plsc.VectorSubcoreMesh(core_axis_name="c", subcore_axis_name="s")
    ch = 4
    cs = bpw // ch

    @functools.partial(
        pl.kernel, mesh=mesh,
        out_type=jax.ShapeDtypeStruct((b, d), jnp.float32),
        scratch_types=[
            pltpu.VMEM((ch, cs), jnp.int32),
            pltpu.VMEM((bpw, d), jnp.float32),
            pltpu.SemaphoreType.DMA((ch,)),
            pltpu.SemaphoreType.DMA((ch,)),
        ],
    )
    def k(table_hbm, idx_hbm, out_hbm, idx_v, rows_v, gsem, osem):
        wid = lax.axis_index("s") * info.num_cores + lax.axis_index("c")
        base = wid * bpw
        pltpu.sync_copy(idx_hbm.at[wid], idx_v)
        gets = [
            pltpu.async_copy(table_hbm.at[idx_v.at[c]],
                             rows_v.at[pl.ds(c * cs, cs)], gsem.at[c])
            for c in range(ch)
        ]
        puts = []
        for c in range(ch):
            gets[c].wait()
            puts.append(pltpu.async_copy(
                rows_v.at[pl.ds(c * cs, cs)],
                out_hbm.at[pl.ds(base + c * cs, cs)], osem.at[c]))
        for p in puts:
            p.wait()

    return k(table, idx.reshape(nw, ch, cs))


def kernel(x, Wq, bq, Wk, bk, Wv, bv, Wo, bo, ln1_g, ln1_b,
           gate_W, gate_b, eW1, eb1, eW2, eb2, ln3_g, ln3_b):
    f32 = jnp.float32
    x2 = x.reshape(S, D)

    gwp = jnp.zeros((D, 128), f32).at[:, :E].set(gate_W)
    gbp = jnp.full((1, 128), -1e30, f32).at[0, :E].set(gate_b)

    whole = lambda shape: pl.BlockSpec(shape, lambda i: tuple(0 for _ in shape))
    x1, eid = pl.pallas_call(
        _attn_body,
        grid=(S // SQ,),
        in_specs=[
            whole((S, D)),
            whole((D, D)), whole((D, D)), whole((D, D)),
            whole((1, D)), whole((1, D)), whole((1, D)),
            whole((D, D)), whole((1, D)),
            whole((1, D)), whole((1, D)),
            whole((D, 128)), whole((1, 128)),
        ],
        out_specs=[
            pl.BlockSpec((SQ, D), lambda i: (i, 0)),
            pl.BlockSpec((SQ, 1), lambda i: (i, 0)),
        ],
        out_shape=[
            jax.ShapeDtypeStruct((S, D), f32),
            jax.ShapeDtypeStruct((S, 1), jnp.int32),
        ],
        scratch_shapes=[
            pltpu.VMEM((S, D), f32),
            pltpu.VMEM((S, D), f32),
        ],
    )(x2, Wq, Wk, Wv, bq.reshape(1, D), bk.reshape(1, D), bv.reshape(1, D),
      Wo, bo.reshape(1, D), ln1_g.reshape(1, D), ln1_b.reshape(1, D),
      gwp, gbp)

    dst, bexp, src = pl.pallas_call(
        _route_body,
        out_shape=[
            jax.ShapeDtypeStruct((S, 1), jnp.int32),
            jax.ShapeDtypeStruct((1, 128), jnp.int32),
            jax.ShapeDtypeStruct((1, G), jnp.int32),
        ],
    )(eid)

    xs = _sc_gather(x1, src.reshape(G))

    ys = pl.pallas_call(
        _ffn_body,
        grid_spec=pltpu.PrefetchScalarGridSpec(
            num_scalar_prefetch=1,
            grid=(NB,),
            in_specs=[
                pl.BlockSpec((BT, D), lambda b, be: (b, 0)),
                pl.BlockSpec((1, D, DFF), lambda b, be: (be[b], 0, 0)),
                pl.BlockSpec((1, 1, DFF), lambda b, be: (be[b], 0, 0)),
                pl.BlockSpec((1, DFF, D), lambda b, be: (be[b], 0, 0)),
                pl.BlockSpec((1, 1, D), lambda b, be: (be[b], 0, 0)),
                pl.BlockSpec((1, D), lambda b, be: (0, 0)),
                pl.BlockSpec((1, D), lambda b, be: (0, 0)),
            ],
            out_specs=pl.BlockSpec((BT, D), lambda b, be: (b, 0)),
        ),
        out_shape=jax.ShapeDtypeStruct((NB * BT, D), f32),
    )(bexp.reshape(128)[:NB], xs, eW1, eb1.reshape(E, 1, DFF),
      eW2, eb2.reshape(E, 1, D),
      ln3_g.reshape(1, D), ln3_b.reshape(1, D))

    out = _sc_gather(ys, dst.reshape(S))
    return out.reshape(1, S, D)
